# baseline (speedup 1.0000x reference)
import functools

import jax
import jax.numpy as jnp
from jax import lax
from jax.experimental import pallas as pl
from jax.experimental.pallas import tpu as pltpu
from jax.experimental.pallas import tpu_sc as plsc

NUM_HEADS = 8
FOUT = 16
FIN = 128
N_NODES = 10000
N_EDGES = 320000

_NC, _NS = 2, 16
_NW = _NC * _NS
_WPH = _NW // NUM_HEADS
_EW = N_EDGES // _WPH
_C = 640
_NCH = _EW // _C
_G = _C // 128
_ROWS_PER_SLAB = N_NODES // 8


def _tc_body(x_ref, w_ref, bs_ref, bt_ref, h_ref, ss_ref, st_ref):
    x = x_ref[...]
    h = lax.dot_general(x, w_ref[...], (((1,), (1,)), ((), ())),
                        preferred_element_type=jnp.float32)
    h_ref[...] = h
    for hd in range(NUM_HEADS):
        slab = h[hd * _ROWS_PER_SLAB:(hd + 1) * _ROWS_PER_SLAB, :]
        ss_ref[hd] = lax.dot_general(slab, bs_ref[hd], (((1,), (0,)), ((), ())),
                                     preferred_element_type=jnp.float32)
        st_ref[hd] = lax.dot_general(slab, bt_ref[hd], (((1,), (0,)), ((), ())),
                                     preferred_element_type=jnp.float32)


def _sc_body(h2, ssrc, stgt, srcx, tgtx, feat_out, sc_out,
             h2s, ssrc_t, stgt_t, srcb0, srcb1, tgtb0, tgtb1, idx0, idx1,
             rows0, rows1, trows0, trows1, sbuf0, sbuf1,
             semi0, semi1, semg0, semg1, semo0, semo1):
    srcb = (srcb0, srcb1)
    tgtb = (tgtb0, tgtb1)
    idx2 = (idx0, idx1)
    rows = (rows0, rows1)
    trows = (trows0, trows1)
    sbuf = (sbuf0, sbuf1)
    semi = (semi0, semi1)
    semg = (semg0, semg1)
    semo = (semo0, semo1)

    cid = lax.axis_index("c")
    sid = lax.axis_index("s")
    hd = cid * (NUM_HEADS // _NC) + lax.bitwise_and(sid, 3)
    sub = lax.shift_right_logical(sid, 2)
    e_base = sub * _EW
    hd_off = lax.bitwise_and(sid, 3) * N_NODES

    half = (NUM_HEADS // _NC) * N_NODES
    stripe = half // _NS
    pltpu.sync_copy(h2.at[pl.ds(cid * half + sid * stripe, stripe)],
                    h2s.at[pl.ds(sid * stripe, stripe)])

    pltpu.sync_copy(ssrc.at[hd], ssrc_t)
    pltpu.sync_copy(stgt.at[hd], stgt_t)
    plsc.subcore_barrier()

    iota16 = lax.iota(jnp.int32, 16)

    def e_off(t):
        return pl.multiple_of(e_base + t * _C, 8)

    def issue_in(t, s):
        eo = e_off(t)
        pltpu.async_copy(srcx.at[pl.ds(eo, _C)], srcb[s], semi[s])
        pltpu.async_copy(tgtx.at[pl.ds(eo, _C)], tgtb[s], semi[s])

    def wait_in(s):
        pltpu.make_async_copy(srcx.at[pl.ds(0, _C)], srcb[s], semi[s]).wait()
        pltpu.make_async_copy(tgtx.at[pl.ds(0, _C)], tgtb[s], semi[s]).wait()

    def compute_idx(s):
        @plsc.parallel_loop(0, _C // 16, unroll=8)
        def _(q):
            idx2[s][pl.ds(q * 16, 16)] = srcb[s][pl.ds(q * 16, 16)] + hd_off

    def fire_gathers(s):
        pltpu.async_copy(h2s.at[idx2[s]], rows[s], semg[s])

    def wait_gathers(s):
        pltpu.make_async_copy(h2.at[pl.ds(0, _C)], rows[s], semg[s]).wait()

    def scores(s):
        @plsc.parallel_loop(0, _C // 16, unroll=16)
        def _(q):
            base = (lax.shift_right_logical(q, 3) * 256
                    + lax.bitwise_and(q, 7) * 16)
            sv = plsc.load_gather(ssrc_t, [srcb[s][pl.ds(q * 16, 16)]])
            sbuf[s][pl.ds(base, 16)] = sv
            tv = plsc.load_gather(stgt_t, [tgtb[s][pl.ds(q * 16, 16)]])
            sbuf[s][pl.ds(base + 128, 16)] = tv

    colv = []
    dvec = []
    for d in range(16):
        fd = lax.bitwise_and(iota16 + d, 15)
        colv.append(fd)
        dvec.append(lax.shift_right_logical(fd, 3) * (_G * 1024)
                    + lax.bitwise_and(fd, 7) * 128 + iota16)

    def transpose_rows(s):
        @plsc.parallel_loop(0, _C // 16, unroll=8)
        def _(b):
            row_idx = iota16 + b * 16
            base_b = (lax.shift_right_logical(b, 3) * 1024
                      + lax.bitwise_and(b, 7) * 16)
            for d in range(16):
                v = plsc.load_gather(rows[s], [row_idx, colv[d]])
                plsc.store_scatter(trows[s], [dvec[d] + base_b], v)

    def issue_out(t, s):
        eo = e_off(t)
        fbase = hd * (16 * N_EDGES) + eo * 8
        pltpu.async_copy(trows[s].at[pl.ds(0, _G * 1024)],
                         feat_out.at[pl.ds(fbase, _G * 1024)], semo[s])
        pltpu.async_copy(trows[s].at[pl.ds(_G * 1024, _G * 1024)],
                         feat_out.at[pl.ds(fbase + 8 * N_EDGES, _G * 1024)],
                         semo[s])
        pltpu.async_copy(sbuf[s],
                         sc_out.at[pl.ds(hd * (2 * N_EDGES) + 2 * eo, 2 * _C)],
                         semo[s])

    def wait_out(s):
        pltpu.make_async_copy(trows[s], feat_out.at[pl.ds(0, 2 * _G * 1024)],
                              semo[s]).wait()
        pltpu.make_async_copy(sbuf[s], sc_out.at[pl.ds(0, 2 * _C)], semo[s]).wait()

    issue_in(0, 0)
    issue_in(1, 1)
    wait_in(0)
    compute_idx(0)
    fire_gathers(0)

    def pair(p, carry):
        wait_in(1)

        @pl.when(p >= 1)
        def _():
            wait_out(1)

        compute_idx(1)
        fire_gathers(1)
        scores(0)
        wait_gathers(0)
        transpose_rows(0)
        issue_out(2 * p, 0)
        issue_in(2 * p + 2, 0)

        wait_in(0)
        wait_out(0)
        compute_idx(0)
        fire_gathers(0)
        scores(1)
        wait_gathers(1)
        transpose_rows(1)
        issue_out(2 * p + 1, 1)

        @pl.when(p < (_NCH - 1) // 2 - 1)
        def _():
            issue_in(2 * p + 3, 1)

        return carry

    lax.fori_loop(0, (_NCH - 1) // 2, pair, 0)

    scores(0)
    wait_gathers(0)
    transpose_rows(0)
    issue_out(_NCH - 1, 0)
    wait_out(0)
    wait_out(1)


def kernel(x, edge_index, W, source_scorer, target_scorer):
    f32 = jnp.float32
    scorer_s = source_scorer.reshape(NUM_HEADS, FOUT)
    scorer_t = target_scorer.reshape(NUM_HEADS, FOUT)
    eye8 = jnp.eye(8, dtype=f32)
    bs = (eye8[None, :, None, :] * scorer_s[:, None, :, None]).reshape(
        NUM_HEADS, FIN, 8)
    bt = (eye8[None, :, None, :] * scorer_t[:, None, :, None]).reshape(
        NUM_HEADS, FIN, 8)

    h, ss, st = pl.pallas_call(
        _tc_body,
        out_shape=[
            jax.ShapeDtypeStruct((N_NODES, FIN), f32),
            jax.ShapeDtypeStruct((NUM_HEADS, _ROWS_PER_SLAB, 8), f32),
            jax.ShapeDtypeStruct((NUM_HEADS, _ROWS_PER_SLAB, 8), f32),
        ],
    )(x, W, bs, bt)

    h2 = h.reshape(NUM_HEADS * N_NODES, FOUT)
    ssrc = ss.reshape(NUM_HEADS, N_NODES)
    stgt = st.reshape(NUM_HEADS, N_NODES)
    src = edge_index[:, 0]
    tgt = edge_index[:, 1]

    mesh = plsc.VectorSubcoreMesh(core_axis_name="c", subcore_axis_name="s")
    sc_call = pl.kernel(
        _sc_body,
        out_type=[
            jax.ShapeDtypeStruct((NUM_HEADS * N_EDGES * FOUT,), f32),
            jax.ShapeDtypeStruct((NUM_HEADS * N_EDGES * 2,), f32),
        ],
        mesh=mesh,
        compiler_params=pltpu.CompilerParams(use_tc_tiling_on_sc=False,
                                             needs_layout_passes=False),
        scratch_types=(
            [pltpu.VMEM_SHARED((NUM_HEADS * N_NODES // _NC, FOUT), f32)]
            + [pltpu.VMEM((N_NODES,), f32)] * 2
            + [pltpu.VMEM((_C,), jnp.int32)] * 4
            + [pltpu.VMEM((_C,), jnp.int32)] * 2
            + [pltpu.VMEM((_C, FOUT), f32)] * 2
            + [pltpu.VMEM((2 * _G * 1024,), f32)] * 2
            + [pltpu.VMEM((2 * _C,), f32)] * 2
            + [pltpu.SemaphoreType.DMA] * 6
        ),
    )
    feat_flat, sc_flat = sc_call(h2, ssrc, stgt, src, tgt)

    selected_features = (
        feat_flat.reshape(NUM_HEADS, 2, N_EDGES // 128, 8, 128)
        .transpose(0, 2, 4, 1, 3)
        .reshape(NUM_HEADS, N_EDGES, FOUT))
    selected_scores = (
        sc_flat.reshape(NUM_HEADS, N_EDGES // 128, 2, 128)
        .transpose(0, 1, 3, 2)
        .reshape(NUM_HEADS, N_EDGES, 2))
    return (selected_features, selected_scores)


# speedup vs baseline: 1.0374x; 1.0374x over previous
import functools

import jax
import jax.numpy as jnp
from jax import lax
from jax.experimental import pallas as pl
from jax.experimental.pallas import tpu as pltpu
from jax.experimental.pallas import tpu_sc as plsc

NUM_HEADS = 8
FOUT = 16
FIN = 128
N_NODES = 10000
N_EDGES = 320000

_NC, _NS = 2, 16
_NW = _NC * _NS
_WPH = _NW // NUM_HEADS
_EW = N_EDGES // _WPH
_C = 640
_NCH = _EW // _C
_G = _C // 128
_ROWS_PER_SLAB = N_NODES // 8


def _tc_body(x_ref, w_ref, bs_ref, bt_ref, h_ref, ss_ref, st_ref):
    x = x_ref[...]
    h = lax.dot_general(x, w_ref[...], (((1,), (1,)), ((), ())),
                        preferred_element_type=jnp.float32)
    h_ref[...] = h
    for hd in range(NUM_HEADS):
        slab = h[hd * _ROWS_PER_SLAB:(hd + 1) * _ROWS_PER_SLAB, :]
        ss_ref[hd] = lax.dot_general(slab, bs_ref[hd], (((1,), (0,)), ((), ())),
                                     preferred_element_type=jnp.float32)
        st_ref[hd] = lax.dot_general(slab, bt_ref[hd], (((1,), (0,)), ((), ())),
                                     preferred_element_type=jnp.float32)


def _sc_body(h2, ssrc, stgt, srcx, tgtx, feat_out, sc_out,
             h2s, ssrc_t, stgt_t, srcb0, srcb1, tgtb0, tgtb1, idx0, idx1,
             rows0, rows1, trows0, trows1, sbuf0, sbuf1,
             semi0, semi1, semg0, semg1, semo0, semo1):
    srcb = (srcb0, srcb1)
    tgtb = (tgtb0, tgtb1)
    idx2 = (idx0, idx1)
    rows = (rows0, rows1)
    trows = (trows0, trows1)
    sbuf = (sbuf0, sbuf1)
    semi = (semi0, semi1)
    semg = (semg0, semg1)
    semo = (semo0, semo1)

    cid = lax.axis_index("c")
    sid = lax.axis_index("s")
    hd = cid * (NUM_HEADS // _NC) + lax.bitwise_and(sid, 3)
    sub = lax.shift_right_logical(sid, 2)
    e_base = sub * _EW
    hd_off = lax.bitwise_and(sid, 3) * N_NODES

    half = (NUM_HEADS // _NC) * N_NODES
    stripe = half // _NS
    pltpu.sync_copy(h2.at[pl.ds(cid * half + sid * stripe, stripe)],
                    h2s.at[pl.ds(sid * stripe, stripe)])

    pltpu.sync_copy(ssrc.at[hd], ssrc_t)
    pltpu.sync_copy(stgt.at[hd], stgt_t)
    plsc.subcore_barrier()

    iota16 = lax.iota(jnp.int32, 16)

    def e_off(t):
        return pl.multiple_of(e_base + t * _C, 8)

    def issue_in(t, s):
        eo = e_off(t)
        pltpu.async_copy(srcx.at[pl.ds(eo, _C)], srcb[s], semi[s])
        pltpu.async_copy(tgtx.at[pl.ds(eo, _C)], tgtb[s], semi[s])

    def wait_in(s):
        pltpu.make_async_copy(srcx.at[pl.ds(0, _C)], srcb[s], semi[s]).wait()
        pltpu.make_async_copy(tgtx.at[pl.ds(0, _C)], tgtb[s], semi[s]).wait()

    def compute_idx(s):
        @plsc.parallel_loop(0, _C // 16, unroll=8)
        def _(q):
            idx2[s][pl.ds(q * 16, 16)] = srcb[s][pl.ds(q * 16, 16)] + hd_off

    def fire_gathers(s):
        pltpu.async_copy(h2s.at[idx2[s]], rows[s], semg[s])

    def wait_gathers(s):
        pltpu.make_async_copy(h2.at[pl.ds(0, _C)], rows[s], semg[s]).wait()

    def scores(s):
        @plsc.parallel_loop(0, _C // 16, unroll=8)
        def _(q):
            base = (lax.shift_right_logical(q, 3) * 256
                    + lax.bitwise_and(q, 7) * 16)
            sv = plsc.load_gather(ssrc_t, [srcb[s][pl.ds(q * 16, 16)]])
            sbuf[s][pl.ds(base, 16)] = sv
            tv = plsc.load_gather(stgt_t, [tgtb[s][pl.ds(q * 16, 16)]])
            sbuf[s][pl.ds(base + 128, 16)] = tv

    colv = []
    dvec = []
    for d in range(16):
        fd = lax.bitwise_and(iota16 + d, 15)
        colv.append(fd)
        dvec.append(lax.shift_right_logical(fd, 3) * (_G * 1024)
                    + lax.bitwise_and(fd, 7) * 128 + iota16)

    def transpose_rows(s):
        @plsc.parallel_loop(0, _C // 16, unroll=4)
        def _(b):
            row_idx = iota16 + b * 16
            base_b = (lax.shift_right_logical(b, 3) * 1024
                      + lax.bitwise_and(b, 7) * 16)
            for d in range(16):
                v = plsc.load_gather(rows[s], [row_idx, colv[d]])
                plsc.store_scatter(trows[s], [dvec[d] + base_b], v)

    def issue_out(t, s):
        eo = e_off(t)
        fbase = hd * (16 * N_EDGES) + eo * 8
        pltpu.async_copy(trows[s].at[pl.ds(0, _G * 1024)],
                         feat_out.at[pl.ds(fbase, _G * 1024)], semo[s])
        pltpu.async_copy(trows[s].at[pl.ds(_G * 1024, _G * 1024)],
                         feat_out.at[pl.ds(fbase + 8 * N_EDGES, _G * 1024)],
                         semo[s])
        pltpu.async_copy(sbuf[s],
                         sc_out.at[pl.ds(hd * (2 * N_EDGES) + 2 * eo, 2 * _C)],
                         semo[s])

    def wait_out(s):
        pltpu.make_async_copy(trows[s], feat_out.at[pl.ds(0, 2 * _G * 1024)],
                              semo[s]).wait()
        pltpu.make_async_copy(sbuf[s], sc_out.at[pl.ds(0, 2 * _C)], semo[s]).wait()

    issue_in(0, 0)
    issue_in(1, 1)
    wait_in(0)
    compute_idx(0)
    fire_gathers(0)

    def pair(p, carry):
        wait_in(1)

        @pl.when(p >= 1)
        def _():
            wait_out(1)

        compute_idx(1)
        fire_gathers(1)
        scores(0)
        wait_gathers(0)
        transpose_rows(0)
        issue_out(2 * p, 0)
        issue_in(2 * p + 2, 0)

        wait_in(0)
        wait_out(0)
        compute_idx(0)
        fire_gathers(0)
        scores(1)
        wait_gathers(1)
        transpose_rows(1)
        issue_out(2 * p + 1, 1)

        @pl.when(p < (_NCH - 1) // 2 - 1)
        def _():
            issue_in(2 * p + 3, 1)

        return carry

    lax.fori_loop(0, (_NCH - 1) // 2, pair, 0)

    scores(0)
    wait_gathers(0)
    transpose_rows(0)
    issue_out(_NCH - 1, 0)
    wait_out(0)
    wait_out(1)


def kernel(x, edge_index, W, source_scorer, target_scorer):
    f32 = jnp.float32
    scorer_s = source_scorer.reshape(NUM_HEADS, FOUT)
    scorer_t = target_scorer.reshape(NUM_HEADS, FOUT)
    eye8 = jnp.eye(8, dtype=f32)
    bs = (eye8[None, :, None, :] * scorer_s[:, None, :, None]).reshape(
        NUM_HEADS, FIN, 8)
    bt = (eye8[None, :, None, :] * scorer_t[:, None, :, None]).reshape(
        NUM_HEADS, FIN, 8)

    h, ss, st = pl.pallas_call(
        _tc_body,
        out_shape=[
            jax.ShapeDtypeStruct((N_NODES, FIN), f32),
            jax.ShapeDtypeStruct((NUM_HEADS, _ROWS_PER_SLAB, 8), f32),
            jax.ShapeDtypeStruct((NUM_HEADS, _ROWS_PER_SLAB, 8), f32),
        ],
    )(x, W, bs, bt)

    h2 = h.reshape(NUM_HEADS * N_NODES, FOUT)
    ssrc = ss.reshape(NUM_HEADS, N_NODES)
    stgt = st.reshape(NUM_HEADS, N_NODES)
    src = edge_index[:, 0]
    tgt = edge_index[:, 1]

    mesh = plsc.VectorSubcoreMesh(core_axis_name="c", subcore_axis_name="s")
    sc_call = pl.kernel(
        _sc_body,
        out_type=[
            jax.ShapeDtypeStruct((NUM_HEADS * N_EDGES * FOUT,), f32),
            jax.ShapeDtypeStruct((NUM_HEADS * N_EDGES * 2,), f32),
        ],
        mesh=mesh,
        compiler_params=pltpu.CompilerParams(use_tc_tiling_on_sc=False,
                                             needs_layout_passes=False),
        scratch_types=(
            [pltpu.VMEM_SHARED((NUM_HEADS * N_NODES // _NC, FOUT), f32)]
            + [pltpu.VMEM((N_NODES,), f32)] * 2
            + [pltpu.VMEM((_C,), jnp.int32)] * 4
            + [pltpu.VMEM((_C,), jnp.int32)] * 2
            + [pltpu.VMEM((_C, FOUT), f32)] * 2
            + [pltpu.VMEM((2 * _G * 1024,), f32)] * 2
            + [pltpu.VMEM((2 * _C,), f32)] * 2
            + [pltpu.SemaphoreType.DMA] * 6
        ),
    )
    feat_flat, sc_flat = sc_call(h2, ssrc, stgt, src, tgt)

    selected_features = (
        feat_flat.reshape(NUM_HEADS, 2, N_EDGES // 128, 8, 128)
        .transpose(0, 2, 4, 1, 3)
        .reshape(NUM_HEADS, N_EDGES, FOUT))
    selected_scores = (
        sc_flat.reshape(NUM_HEADS, N_EDGES // 128, 2, 128)
        .transpose(0, 1, 3, 2)
        .reshape(NUM_HEADS, N_EDGES, 2))
    return (selected_features, selected_scores)
